# baseline (speedup 1.0000x reference)
import functools

import jax
import jax.numpy as jnp
from jax import lax
from jax.experimental import pallas as pl
from jax.experimental.pallas import tpu as pltpu
from jax.experimental.pallas import tpu_sc as plsc

NUM_USERS = 1000000
NUM_NURSE = 100000
EMBED = 32
BATCH = 16384

_NC = 2
_NS = 16
_NW = _NC * _NS
_BPW = BATCH // _NW
_CHUNK = 128
_NCHUNK = _BPW // _CHUNK


def _sc_body(uidx, nidx, uemb, nemb, ubias, nbias,
             part_out, bsum_out,
             idx_u, idx_n, u_rows, n_rows, ub_v, nb_v, bs_v, acc_v, sem):
    wid = lax.axis_index("s") * _NC + lax.axis_index("c")
    base = wid * _BPW

    pltpu.sync_copy(uidx.at[wid], idx_u)
    pltpu.sync_copy(nidx.at[wid], idx_n)

    copies = []
    for k in range(_NCHUNK):
        sl = pl.ds(k * _CHUNK, _CHUNK)
        copies.append(pltpu.async_copy(uemb.at[idx_u.at[k]], u_rows.at[sl], sem))
        copies.append(pltpu.async_copy(nemb.at[idx_n.at[k]], n_rows.at[sl], sem))
        copies.append(pltpu.async_copy(ubias.at[idx_u.at[k]], ub_v.at[sl], sem))
        copies.append(pltpu.async_copy(nbias.at[idx_n.at[k]], nb_v.at[sl], sem))
    for c in copies:
        c.wait()

    def dot_body(i, acc):
        a = u_rows[i, pl.ds(0, 16)] * n_rows[i, pl.ds(0, 16)]
        b = u_rows[i, pl.ds(16, 16)] * n_rows[i, pl.ds(16, 16)]
        return acc + a + b

    acc = lax.fori_loop(0, _BPW, dot_body, jnp.zeros((16,), jnp.float32))
    acc_v[...] = acc
    pltpu.sync_copy(acc_v, part_out.at[wid])

    def bias_body(j, carry):
        sl = pl.ds(j * 16, 16)
        bs_v[sl] = ub_v[sl] + nb_v[sl]
        return carry

    lax.fori_loop(0, _BPW // 16, bias_body, 0)
    pltpu.sync_copy(bs_v, bsum_out.at[pl.ds(base, _BPW)])


@functools.partial(jax.jit, static_argnums=())
def _sc_gather_dot(uidx, nidx, uemb, nemb, ubias, nbias):
    mesh = plsc.VectorSubcoreMesh(core_axis_name="c", subcore_axis_name="s")
    kfn = pl.kernel(
        _sc_body,
        out_type=[
            jax.ShapeDtypeStruct((_NW, 16), jnp.float32),
            jax.ShapeDtypeStruct((BATCH,), jnp.float32),
        ],
        mesh=mesh,
        compiler_params=pltpu.CompilerParams(use_tc_tiling_on_sc=False),
        scratch_types=[
            pltpu.VMEM((_NCHUNK, _CHUNK), jnp.int32),
            pltpu.VMEM((_NCHUNK, _CHUNK), jnp.int32),
            pltpu.VMEM((_BPW, EMBED), jnp.float32),
            pltpu.VMEM((_BPW, EMBED), jnp.float32),
            pltpu.VMEM((_BPW,), jnp.float32),
            pltpu.VMEM((_BPW,), jnp.float32),
            pltpu.VMEM((_BPW,), jnp.float32),
            pltpu.VMEM((16,), jnp.float32),
            pltpu.SemaphoreType.DMA,
        ],
    )
    return kfn(uidx, nidx, uemb, nemb, ubias, nbias)


def _tc_body(part_ref, x_ref, o_ref):
    s = jnp.sum(part_ref[...])
    o_ref[...] = jax.nn.sigmoid(x_ref[...] + s)


def _tc_finish(partials, bsum2d):
    return pl.pallas_call(
        _tc_body,
        out_shape=jax.ShapeDtypeStruct((128, 128), jnp.float32),
    )(partials, bsum2d)


def kernel(inputs, user_embedding, nurse_embedding, user_bias, nurse_bias):
    uidx = inputs[:, 0].astype(jnp.int32).reshape(_NW, _NCHUNK, _CHUNK)
    nidx = inputs[:, 1].astype(jnp.int32).reshape(_NW, _NCHUNK, _CHUNK)
    ubias = user_bias.reshape(-1)
    nbias = nurse_bias.reshape(-1)
    partials, bsum = _sc_gather_dot(uidx, nidx, user_embedding,
                                    nurse_embedding, ubias, nbias)
    out = _tc_finish(partials, bsum.reshape(128, 128))
    return out.reshape(BATCH, 1)
